# baseline (speedup 1.0000x reference)
import functools

import jax
import jax.numpy as jnp
from jax import lax
from jax.experimental import pallas as pl
from jax.experimental.pallas import tpu as pltpu
from jax.experimental.pallas import tpu_sc as plsc

NN = 10000
EE = 320000
DD = 128
OO = 10
NP = 10240
NC, NS, LN = 2, 16, 16
NW = NC * NS
CH = 128
JC = 80
TOTC = NW * JC
EP = TOTC * CH
CR = CH
JC0 = 56
JC1 = 104
RPT = NP // NS
NBUF = 2

_mesh = plsc.VectorSubcoreMesh(
    core_axis_name="c", subcore_axis_name="s", num_cores=NC, num_subcores=NS)


_RCH = [(k * CR, CR) for k in range(RPT // CR)]
if RPT % CR:
    _RCH.append((RPT - RPT % CR, RPT % CR))


@functools.partial(
    pl.kernel,
    out_type=jax.ShapeDtypeStruct((NC, NP, DD), jnp.float32),
    mesh=_mesh,
    scratch_types=[
        pltpu.VMEM_SHARED((NP, DD), jnp.float32),
        pltpu.VMEM((JC, CH), jnp.int32),
        pltpu.VMEM((CR, DD), jnp.float32),
        pltpu.SemaphoreType.DMA,
    ],
)
def _sc_degree(dst2, deg_out, dacc, dstv, ones, sem):
    c = lax.axis_index("c")
    t = lax.axis_index("s")
    w = c * NS + t

    def _z(r, _):
        for k in range(DD // LN):
            ones[r, pl.ds(k * LN, LN)] = jnp.zeros((LN,), jnp.float32)
        return _
    lax.fori_loop(0, CR, _z, None)
    zh = [pltpu.async_copy(ones.at[pl.ds(0, n)],
                           dacc.at[pl.ds(t * RPT + o, n)], sem)
          for o, n in _RCH]
    for h in zh:
        h.wait()

    def _o(r, _):
        for k in range(DD // LN):
            ones[r, pl.ds(k * LN, LN)] = jnp.ones((LN,), jnp.float32)
        return _
    lax.fori_loop(0, CR, _o, None)
    plsc.subcore_barrier()

    pltpu.sync_copy(dst2.at[pl.ds(w * JC, JC)], dstv)
    handles = [
        pltpu.async_copy(ones, dacc.at[dstv.at[j]], sem, add=True)
        for j in range(JC)
    ]
    for h in handles:
        h.wait()
    plsc.subcore_barrier()

    dh = [pltpu.async_copy(dacc.at[pl.ds(t * RPT + o, n)],
                           deg_out.at[c, pl.ds(t * RPT + o, n)], sem)
          for o, n in _RCH]
    for h in dh:
        h.wait()


IB = 4
JCM = max(JC0, JC1)


@functools.partial(
    pl.kernel,
    out_type=jax.ShapeDtypeStruct((NC, NP, DD), jnp.float32),
    mesh=_mesh,
    scratch_types=[
        pltpu.VMEM_SHARED((NP, DD), jnp.float32),
        pltpu.VMEM((JCM, CH), jnp.int32),
        pltpu.VMEM((IB, CH), jnp.int32),
        [pltpu.VMEM((CH, DD), jnp.float32) for _ in range(2)],
        [pltpu.SemaphoreType.DMA for _ in range(2)],
        [pltpu.SemaphoreType.DMA for _ in range(IB)],
        [pltpu.SemaphoreType.DMA for _ in range(2)],
        pltpu.SemaphoreType.DMA,
    ],
)
def _sc_aggregate(table, srcf, dst2, out, acc, dstv, ibuf, bufs, gsems, isems,
                  ssems, dsem):
    c = lax.axis_index("c")
    t = lax.axis_index("s")

    def _z(r, _):
        for k in range(DD // LN):
            bufs[0][r, pl.ds(k * LN, LN)] = jnp.zeros((LN,), jnp.float32)
        return _
    lax.fori_loop(0, CR, _z, None)
    zh = [pltpu.async_copy(bufs[0].at[pl.ds(0, n)],
                           acc.at[pl.ds(t * RPT + o, n)], dsem)
          for o, n in _RCH]
    for h in zh:
        h.wait()
    plsc.subcore_barrier()

    def _run(jc, base):
        pltpu.sync_copy(dst2.at[pl.ds(base, jc)], dstv.at[pl.ds(0, jc)])

        def _idx_copy(j):
            return pltpu.async_copy(srcf.at[pl.ds((base + j) * CH, CH)],
                                    ibuf.at[j % IB], isems[j % IB])

        def _gather(j):
            return pltpu.async_copy(table.at[ibuf.at[j % IB]],
                                    bufs[j % 2], gsems[j % 2])

        def _scatter(j):
            return pltpu.async_copy(bufs[j % 2], acc.at[dstv.at[j]],
                                    ssems[j % 2], add=True)

        ih, gh, sh = {}, {}, {}
        for j in range(min(IB, jc)):
            ih[j] = _idx_copy(j)
        ih[0].wait()
        gh[0] = _gather(0)
        for j in range(jc):
            if j > 0:
                sh[j - 1].wait()
            nj = j + 1
            if nj < jc:
                ih[nj].wait()
                gh[nj] = _gather(nj)
            gh[j].wait()
            if j + IB < jc:
                ih[j + IB] = _idx_copy(j + IB)
            sh[j] = _scatter(j)
        sh[jc - 1].wait()

    @pl.when(c == 0)
    def _():
        _run(JC0, t * JC0)

    @pl.when(c == 1)
    def _():
        _run(JC1, NS * JC0 + t * JC1)

    plsc.subcore_barrier()

    dh = [pltpu.async_copy(acc.at[pl.ds(t * RPT + o, n)],
                           out.at[c, pl.ds(t * RPT + o, n)], dsem)
          for o, n in _RCH]
    for h in dh:
        h.wait()


_GRID = 8
_BR = NP // _GRID


def _row_spec():
    return pl.BlockSpec((_BR, DD), lambda i: (i, 0))


def _full_spec(shape):
    return pl.BlockSpec(shape, lambda i: tuple(0 for _ in shape))


def _tc_pre_body(x_ref, w_ref, deg_ref, o_ref, dinv_ref):
    i = pl.program_id(0)
    deg = 1.0 + deg_ref[0] + deg_ref[1]
    row = i * _BR + jax.lax.broadcasted_iota(jnp.int32, (_BR, DD), 0)
    dinv = jnp.where(row < NN, jax.lax.rsqrt(deg), 0.0)
    dinv_ref[...] = dinv
    o_ref[...] = dinv * jnp.dot(
        x_ref[...], w_ref[...], preferred_element_type=jnp.float32)


def _tc_pre(xp, w0, degp):
    return pl.pallas_call(
        _tc_pre_body,
        grid=(_GRID,),
        in_specs=[
            _row_spec(),
            _full_spec((DD, DD)),
            pl.BlockSpec((NC, _BR, DD), lambda i: (0, i, 0)),
        ],
        out_specs=(_row_spec(), _row_spec()),
        out_shape=(jax.ShapeDtypeStruct((NP, DD), jnp.float32),
                   jax.ShapeDtypeStruct((NP, DD), jnp.float32)),
    )(xp, w0, degp)


def _tc_mid_body(h_ref, s_ref, dinv_ref, b_ref, w_ref, o_ref):
    dinv = dinv_ref[...]
    u = jax.nn.relu(b_ref[...] + dinv * (h_ref[...] + s_ref[0] + s_ref[1]))
    o_ref[...] = dinv * jnp.dot(u, w_ref[...],
                                preferred_element_type=jnp.float32)


def _tc_mid(hp, s, dinv, b, w):
    return pl.pallas_call(
        _tc_mid_body,
        grid=(_GRID,),
        in_specs=[
            _row_spec(),
            pl.BlockSpec((NC, _BR, DD), lambda i: (0, i, 0)),
            _row_spec(),
            _full_spec((1, DD)),
            _full_spec((DD, DD)),
        ],
        out_specs=_row_spec(),
        out_shape=jax.ShapeDtypeStruct((NP, DD), jnp.float32),
    )(hp, s, dinv, b, w)


def _tc_post_body(h_ref, s_ref, dinv_ref, b_ref, w_ref, bl_ref, o_ref):
    u = jax.nn.relu(
        b_ref[...] + dinv_ref[...] * (h_ref[...] + s_ref[0] + s_ref[1]))
    o_ref[...] = jnp.dot(u, w_ref[...],
                         preferred_element_type=jnp.float32) + bl_ref[...]


def _tc_post(hp, s, dinv, b, w, bl):
    return pl.pallas_call(
        _tc_post_body,
        grid=(_GRID,),
        in_specs=[
            _row_spec(),
            pl.BlockSpec((NC, _BR, DD), lambda i: (0, i, 0)),
            _row_spec(),
            _full_spec((1, DD)),
            _full_spec((DD, DD)),
            _full_spec((1, DD)),
        ],
        out_specs=_row_spec(),
        out_shape=jax.ShapeDtypeStruct((NP, DD), jnp.float32),
    )(hp, s, dinv, b, w, bl)


def kernel(x, edge_index, W0, b0, W1, b1, W2, b2, Wlin, blin):
    src = edge_index[0].astype(jnp.int32)
    dst = edge_index[1].astype(jnp.int32)
    pad = jnp.full((EP - EE,), NN, jnp.int32)
    srcf = jnp.concatenate([src, pad])
    dst2 = jnp.concatenate([dst, pad]).reshape(TOTC, CH)
    xp = jnp.pad(x, ((0, NP - NN), (0, 0)))
    wlp = jnp.pad(Wlin, ((0, 0), (0, DD - OO)))
    blp = jnp.pad(blin, (0, DD - OO)).reshape(1, DD)

    degp = _sc_degree(dst2)
    h0, dinv = _tc_pre(xp, W0, degp)
    s0 = _sc_aggregate(h0, srcf, dst2)
    h1 = _tc_mid(h0, s0, dinv, b0.reshape(1, DD), W1)
    s1 = _sc_aggregate(h1, srcf, dst2)
    h2 = _tc_mid(h1, s1, dinv, b1.reshape(1, DD), W2)
    s2 = _sc_aggregate(h2, srcf, dst2)
    outp = _tc_post(h2, s2, dinv, b2.reshape(1, DD), wlp, blp)
    return outp[:NN, :OO]


# speedup vs baseline: 1.0378x; 1.0378x over previous
import functools

import jax
import jax.numpy as jnp
from jax import lax
from jax.experimental import pallas as pl
from jax.experimental.pallas import tpu as pltpu
from jax.experimental.pallas import tpu_sc as plsc

NN = 10000
EE = 320000
DD = 128
OO = 10
NP = 10240
NC, NS, LN = 2, 16, 16
NW = NC * NS
CH = 128
JC = 80
TOTC = NW * JC
EP = TOTC * CH
CR = CH
JC0 = 80
JC1 = 80
RPT = NP // NS
NBUF = 2

_mesh = plsc.VectorSubcoreMesh(
    core_axis_name="c", subcore_axis_name="s", num_cores=NC, num_subcores=NS)


_RCH = [(k * CR, CR) for k in range(RPT // CR)]
if RPT % CR:
    _RCH.append((RPT - RPT % CR, RPT % CR))


@functools.partial(
    pl.kernel,
    out_type=jax.ShapeDtypeStruct((NC, NP, DD), jnp.float32),
    mesh=_mesh,
    scratch_types=[
        pltpu.VMEM_SHARED((NP, DD), jnp.float32),
        pltpu.VMEM((JC, CH), jnp.int32),
        pltpu.VMEM((CR, DD), jnp.float32),
        pltpu.SemaphoreType.DMA,
    ],
)
def _sc_degree(dst2, deg_out, dacc, dstv, ones, sem):
    c = lax.axis_index("c")
    t = lax.axis_index("s")
    w = c * NS + t

    def _z(r, _):
        for k in range(DD // LN):
            ones[r, pl.ds(k * LN, LN)] = jnp.zeros((LN,), jnp.float32)
        return _
    lax.fori_loop(0, CR, _z, None)
    zh = [pltpu.async_copy(ones.at[pl.ds(0, n)],
                           dacc.at[pl.ds(t * RPT + o, n)], sem)
          for o, n in _RCH]
    for h in zh:
        h.wait()

    def _o(r, _):
        for k in range(DD // LN):
            ones[r, pl.ds(k * LN, LN)] = jnp.ones((LN,), jnp.float32)
        return _
    lax.fori_loop(0, CR, _o, None)
    plsc.subcore_barrier()

    pltpu.sync_copy(dst2.at[pl.ds(w * JC, JC)], dstv)
    handles = [
        pltpu.async_copy(ones, dacc.at[dstv.at[j]], sem, add=True)
        for j in range(JC)
    ]
    for h in handles:
        h.wait()
    plsc.subcore_barrier()

    dh = [pltpu.async_copy(dacc.at[pl.ds(t * RPT + o, n)],
                           deg_out.at[c, pl.ds(t * RPT + o, n)], sem)
          for o, n in _RCH]
    for h in dh:
        h.wait()


IB = 4
JCM = max(JC0, JC1)


@functools.partial(
    pl.kernel,
    out_type=jax.ShapeDtypeStruct((NC, NP, DD), jnp.float32),
    mesh=_mesh,
    scratch_types=[
        pltpu.VMEM_SHARED((NP, DD), jnp.float32),
        pltpu.VMEM((JCM, CH), jnp.int32),
        pltpu.VMEM((IB, CH), jnp.int32),
        [pltpu.VMEM((CH, DD), jnp.float32) for _ in range(2)],
        [pltpu.SemaphoreType.DMA for _ in range(2)],
        [pltpu.SemaphoreType.DMA for _ in range(IB)],
        [pltpu.SemaphoreType.DMA for _ in range(2)],
        pltpu.SemaphoreType.DMA,
    ],
)
def _sc_aggregate(table, srcf, dst2, out, acc, dstv, ibuf, bufs, gsems, isems,
                  ssems, dsem):
    c = lax.axis_index("c")
    t = lax.axis_index("s")

    def _z(r, _):
        for k in range(DD // LN):
            bufs[0][r, pl.ds(k * LN, LN)] = jnp.zeros((LN,), jnp.float32)
        return _
    lax.fori_loop(0, CR, _z, None)
    zh = [pltpu.async_copy(bufs[0].at[pl.ds(0, n)],
                           acc.at[pl.ds(t * RPT + o, n)], dsem)
          for o, n in _RCH]
    for h in zh:
        h.wait()
    plsc.subcore_barrier()

    def _run(jc, base):
        pltpu.sync_copy(dst2.at[pl.ds(base, jc)], dstv.at[pl.ds(0, jc)])

        def _idx_copy(j):
            return pltpu.async_copy(srcf.at[pl.ds((base + j) * CH, CH)],
                                    ibuf.at[j % IB], isems[j % IB])

        def _gather(j):
            return pltpu.async_copy(table.at[ibuf.at[j % IB]],
                                    bufs[j % 2], gsems[j % 2])

        def _scatter(j):
            return pltpu.async_copy(bufs[j % 2], acc.at[dstv.at[j]],
                                    ssems[j % 2], add=True)

        ih, gh, sh = {}, {}, {}
        for j in range(min(IB, jc)):
            ih[j] = _idx_copy(j)
        ih[0].wait()
        gh[0] = _gather(0)
        for j in range(jc):
            if j > 0:
                sh[j - 1].wait()
            nj = j + 1
            if nj < jc:
                ih[nj].wait()
                gh[nj] = _gather(nj)
            gh[j].wait()
            if j + IB < jc:
                ih[j + IB] = _idx_copy(j + IB)
            sh[j] = _scatter(j)
        sh[jc - 1].wait()

    @pl.when(c == 0)
    def _():
        _run(JC0, t * JC0)

    @pl.when(c == 1)
    def _():
        _run(JC1, NS * JC0 + t * JC1)

    plsc.subcore_barrier()

    dh = [pltpu.async_copy(acc.at[pl.ds(t * RPT + o, n)],
                           out.at[c, pl.ds(t * RPT + o, n)], dsem)
          for o, n in _RCH]
    for h in dh:
        h.wait()


_GRID = 8
_BR = NP // _GRID


def _row_spec():
    return pl.BlockSpec((_BR, DD), lambda i: (i, 0))


def _full_spec(shape):
    return pl.BlockSpec(shape, lambda i: tuple(0 for _ in shape))


def _tc_pre_body(x_ref, w_ref, deg_ref, o_ref, dinv_ref):
    i = pl.program_id(0)
    deg = 1.0 + deg_ref[0] + deg_ref[1]
    row = i * _BR + jax.lax.broadcasted_iota(jnp.int32, (_BR, DD), 0)
    dinv = jnp.where(row < NN, jax.lax.rsqrt(deg), 0.0)
    dinv_ref[...] = dinv
    o_ref[...] = dinv * jnp.dot(
        x_ref[...], w_ref[...], preferred_element_type=jnp.float32)


def _tc_pre(xp, w0, degp):
    return pl.pallas_call(
        _tc_pre_body,
        grid=(_GRID,),
        in_specs=[
            _row_spec(),
            _full_spec((DD, DD)),
            pl.BlockSpec((NC, _BR, DD), lambda i: (0, i, 0)),
        ],
        out_specs=(_row_spec(), _row_spec()),
        out_shape=(jax.ShapeDtypeStruct((NP, DD), jnp.float32),
                   jax.ShapeDtypeStruct((NP, DD), jnp.float32)),
    )(xp, w0, degp)


def _tc_mid_body(h_ref, s_ref, dinv_ref, b_ref, w_ref, o_ref):
    dinv = dinv_ref[...]
    u = jax.nn.relu(b_ref[...] + dinv * (h_ref[...] + s_ref[0] + s_ref[1]))
    o_ref[...] = dinv * jnp.dot(u, w_ref[...],
                                preferred_element_type=jnp.float32)


def _tc_mid(hp, s, dinv, b, w):
    return pl.pallas_call(
        _tc_mid_body,
        grid=(_GRID,),
        in_specs=[
            _row_spec(),
            pl.BlockSpec((NC, _BR, DD), lambda i: (0, i, 0)),
            _row_spec(),
            _full_spec((1, DD)),
            _full_spec((DD, DD)),
        ],
        out_specs=_row_spec(),
        out_shape=jax.ShapeDtypeStruct((NP, DD), jnp.float32),
    )(hp, s, dinv, b, w)


def _tc_post_body(h_ref, s_ref, dinv_ref, b_ref, w_ref, bl_ref, o_ref):
    u = jax.nn.relu(
        b_ref[...] + dinv_ref[...] * (h_ref[...] + s_ref[0] + s_ref[1]))
    o_ref[...] = jnp.dot(u, w_ref[...],
                         preferred_element_type=jnp.float32) + bl_ref[...]


def _tc_post(hp, s, dinv, b, w, bl):
    return pl.pallas_call(
        _tc_post_body,
        grid=(_GRID,),
        in_specs=[
            _row_spec(),
            pl.BlockSpec((NC, _BR, DD), lambda i: (0, i, 0)),
            _row_spec(),
            _full_spec((1, DD)),
            _full_spec((DD, DD)),
            _full_spec((1, DD)),
        ],
        out_specs=_row_spec(),
        out_shape=jax.ShapeDtypeStruct((NP, DD), jnp.float32),
    )(hp, s, dinv, b, w, bl)


def kernel(x, edge_index, W0, b0, W1, b1, W2, b2, Wlin, blin):
    src = edge_index[0].astype(jnp.int32)
    dst = edge_index[1].astype(jnp.int32)
    pad = jnp.full((EP - EE,), NN, jnp.int32)
    srcf = jnp.concatenate([src, pad])
    dst2 = jnp.concatenate([dst, pad]).reshape(TOTC, CH)
    xp = jnp.pad(x, ((0, NP - NN), (0, 0)))
    wlp = jnp.pad(Wlin, ((0, 0), (0, DD - OO)))
    blp = jnp.pad(blin, (0, DD - OO)).reshape(1, DD)

    degp = _sc_degree(dst2)
    h0, dinv = _tc_pre(xp, W0, degp)
    s0 = _sc_aggregate(h0, srcf, dst2)
    h1 = _tc_mid(h0, s0, dinv, b0.reshape(1, DD), W1)
    s1 = _sc_aggregate(h1, srcf, dst2)
    h2 = _tc_mid(h1, s1, dinv, b1.reshape(1, DD), W2)
    s2 = _sc_aggregate(h2, srcf, dst2)
    outp = _tc_post(h2, s2, dinv, b2.reshape(1, DD), wlp, blp)
    return outp[:NN, :OO]


# speedup vs baseline: 1.0385x; 1.0007x over previous
import functools

import jax
import jax.numpy as jnp
from jax import lax
from jax.experimental import pallas as pl
from jax.experimental.pallas import tpu as pltpu
from jax.experimental.pallas import tpu_sc as plsc

NN = 10000
EE = 320000
DD = 128
OO = 10
NP = 10240
NC, NS, LN = 2, 16, 16
NW = NC * NS
CH = 128
JC = 80
TOTC = NW * JC
EP = TOTC * CH
CR = CH
JC0 = 80
JC1 = 80
RPT = NP // NS
NBUF = 2

_mesh = plsc.VectorSubcoreMesh(
    core_axis_name="c", subcore_axis_name="s", num_cores=NC, num_subcores=NS)


_RCH = [(k * CR, CR) for k in range(RPT // CR)]
if RPT % CR:
    _RCH.append((RPT - RPT % CR, RPT % CR))


@functools.partial(
    pl.kernel,
    out_type=jax.ShapeDtypeStruct((NC, NP, DD), jnp.float32),
    mesh=_mesh,
    scratch_types=[
        pltpu.VMEM_SHARED((NP, DD), jnp.float32),
        pltpu.VMEM((JC, CH), jnp.int32),
        pltpu.VMEM((CR, DD), jnp.float32),
        pltpu.SemaphoreType.DMA,
    ],
)
def _sc_degree(dst2, deg_out, dacc, dstv, ones, sem):
    c = lax.axis_index("c")
    t = lax.axis_index("s")
    w = c * NS + t

    def _z(r, _):
        for k in range(DD // LN):
            ones[r, pl.ds(k * LN, LN)] = jnp.zeros((LN,), jnp.float32)
        return _
    lax.fori_loop(0, CR, _z, None)
    zh = [pltpu.async_copy(ones.at[pl.ds(0, n)],
                           dacc.at[pl.ds(t * RPT + o, n)], sem)
          for o, n in _RCH]
    for h in zh:
        h.wait()

    def _o(r, _):
        for k in range(DD // LN):
            ones[r, pl.ds(k * LN, LN)] = jnp.ones((LN,), jnp.float32)
        return _
    lax.fori_loop(0, CR, _o, None)
    plsc.subcore_barrier()

    pltpu.sync_copy(dst2.at[pl.ds(w * JC, JC)], dstv)
    handles = [
        pltpu.async_copy(ones, dacc.at[dstv.at[j]], sem, add=True)
        for j in range(JC)
    ]
    for h in handles:
        h.wait()
    plsc.subcore_barrier()

    dh = [pltpu.async_copy(dacc.at[pl.ds(t * RPT + o, n)],
                           deg_out.at[c, pl.ds(t * RPT + o, n)], sem)
          for o, n in _RCH]
    for h in dh:
        h.wait()


IB = 4
JCM = max(JC0, JC1)


@functools.partial(
    pl.kernel,
    out_type=jax.ShapeDtypeStruct((NC, NP, DD), jnp.float32),
    mesh=_mesh,
    scratch_types=[
        pltpu.VMEM_SHARED((NP, DD), jnp.float32),
        pltpu.VMEM((JCM, CH), jnp.int32),
        pltpu.VMEM((IB, CH), jnp.int32),
        [pltpu.VMEM((CH, DD), jnp.float32) for _ in range(2)],
        [pltpu.SemaphoreType.DMA for _ in range(2)],
        [pltpu.SemaphoreType.DMA for _ in range(IB)],
        [pltpu.SemaphoreType.DMA for _ in range(2)],
        pltpu.SemaphoreType.DMA,
    ],
)
def _sc_aggregate(table, srcf, dst2, out, acc, dstv, ibuf, bufs, gsems, isems,
                  ssems, dsem):
    c = lax.axis_index("c")
    t = lax.axis_index("s")

    def _z(r, _):
        for k in range(DD // LN):
            bufs[0][r, pl.ds(k * LN, LN)] = jnp.zeros((LN,), jnp.float32)
        return _
    lax.fori_loop(0, CR, _z, None)
    zh = [pltpu.async_copy(bufs[0].at[pl.ds(0, n)],
                           acc.at[pl.ds(t * RPT + o, n)], dsem)
          for o, n in _RCH]
    for h in zh:
        h.wait()
    plsc.subcore_barrier()

    def _run(jc, base):
        pltpu.sync_copy(dst2.at[pl.ds(base, jc)], dstv.at[pl.ds(0, jc)])

        def _idx_copy(j):
            return pltpu.async_copy(srcf.at[pl.ds((base + j) * CH, CH)],
                                    ibuf.at[j % IB], isems[j % IB])

        def _gather(j):
            return pltpu.async_copy(table.at[ibuf.at[j % IB]],
                                    bufs[j % 2], gsems[j % 2])

        def _scatter(j):
            return pltpu.async_copy(bufs[j % 2], acc.at[dstv.at[j]],
                                    ssems[j % 2], add=True)

        ih, gh, sh = {}, {}, {}
        for j in range(min(IB, jc)):
            ih[j] = _idx_copy(j)
        ih[0].wait()
        gh[0] = _gather(0)
        for j in range(jc):
            if j > 0:
                sh[j - 1].wait()
            nj = j + 1
            if nj < jc:
                ih[nj].wait()
                gh[nj] = _gather(nj)
            gh[j].wait()
            if j + IB < jc:
                ih[j + IB] = _idx_copy(j + IB)
            sh[j] = _scatter(j)
        sh[jc - 1].wait()

    _run(JC0, (c * NS + t) * JC0)

    plsc.subcore_barrier()

    dh = [pltpu.async_copy(acc.at[pl.ds(t * RPT + o, n)],
                           out.at[c, pl.ds(t * RPT + o, n)], dsem)
          for o, n in _RCH]
    for h in dh:
        h.wait()


_GRID = 8
_BR = NP // _GRID


def _row_spec():
    return pl.BlockSpec((_BR, DD), lambda i: (i, 0))


def _full_spec(shape):
    return pl.BlockSpec(shape, lambda i: tuple(0 for _ in shape))


def _tc_pre_body(x_ref, w_ref, deg_ref, o_ref, dinv_ref):
    i = pl.program_id(0)
    deg = 1.0 + deg_ref[0] + deg_ref[1]
    row = i * _BR + jax.lax.broadcasted_iota(jnp.int32, (_BR, DD), 0)
    dinv = jnp.where(row < NN, jax.lax.rsqrt(deg), 0.0)
    dinv_ref[...] = dinv
    o_ref[...] = dinv * jnp.dot(
        x_ref[...], w_ref[...], preferred_element_type=jnp.float32)


def _tc_pre(xp, w0, degp):
    return pl.pallas_call(
        _tc_pre_body,
        grid=(_GRID,),
        in_specs=[
            _row_spec(),
            _full_spec((DD, DD)),
            pl.BlockSpec((NC, _BR, DD), lambda i: (0, i, 0)),
        ],
        out_specs=(_row_spec(), _row_spec()),
        out_shape=(jax.ShapeDtypeStruct((NP, DD), jnp.float32),
                   jax.ShapeDtypeStruct((NP, DD), jnp.float32)),
    )(xp, w0, degp)


def _tc_mid_body(h_ref, s_ref, dinv_ref, b_ref, w_ref, o_ref):
    dinv = dinv_ref[...]
    u = jax.nn.relu(b_ref[...] + dinv * (h_ref[...] + s_ref[0] + s_ref[1]))
    o_ref[...] = dinv * jnp.dot(u, w_ref[...],
                                preferred_element_type=jnp.float32)


def _tc_mid(hp, s, dinv, b, w):
    return pl.pallas_call(
        _tc_mid_body,
        grid=(_GRID,),
        in_specs=[
            _row_spec(),
            pl.BlockSpec((NC, _BR, DD), lambda i: (0, i, 0)),
            _row_spec(),
            _full_spec((1, DD)),
            _full_spec((DD, DD)),
        ],
        out_specs=_row_spec(),
        out_shape=jax.ShapeDtypeStruct((NP, DD), jnp.float32),
    )(hp, s, dinv, b, w)


def _tc_post_body(h_ref, s_ref, dinv_ref, b_ref, w_ref, bl_ref, o_ref):
    u = jax.nn.relu(
        b_ref[...] + dinv_ref[...] * (h_ref[...] + s_ref[0] + s_ref[1]))
    o_ref[...] = jnp.dot(u, w_ref[...],
                         preferred_element_type=jnp.float32) + bl_ref[...]


def _tc_post(hp, s, dinv, b, w, bl):
    return pl.pallas_call(
        _tc_post_body,
        grid=(_GRID,),
        in_specs=[
            _row_spec(),
            pl.BlockSpec((NC, _BR, DD), lambda i: (0, i, 0)),
            _row_spec(),
            _full_spec((1, DD)),
            _full_spec((DD, DD)),
            _full_spec((1, DD)),
        ],
        out_specs=_row_spec(),
        out_shape=jax.ShapeDtypeStruct((NP, DD), jnp.float32),
    )(hp, s, dinv, b, w, bl)


def kernel(x, edge_index, W0, b0, W1, b1, W2, b2, Wlin, blin):
    src = edge_index[0].astype(jnp.int32)
    dst = edge_index[1].astype(jnp.int32)
    pad = jnp.full((EP - EE,), NN, jnp.int32)
    srcf = jnp.concatenate([src, pad])
    dst2 = jnp.concatenate([dst, pad]).reshape(TOTC, CH)
    xp = jnp.pad(x, ((0, NP - NN), (0, 0)))
    wlp = jnp.pad(Wlin, ((0, 0), (0, DD - OO)))
    blp = jnp.pad(blin, (0, DD - OO)).reshape(1, DD)

    degp = _sc_degree(dst2)
    h0, dinv = _tc_pre(xp, W0, degp)
    s0 = _sc_aggregate(h0, srcf, dst2)
    h1 = _tc_mid(h0, s0, dinv, b0.reshape(1, DD), W1)
    s1 = _sc_aggregate(h1, srcf, dst2)
    h2 = _tc_mid(h1, s1, dinv, b1.reshape(1, DD), W2)
    s2 = _sc_aggregate(h2, srcf, dst2)
    outp = _tc_post(h2, s2, dinv, b2.reshape(1, DD), wlp, blp)
    return outp[:NN, :OO]


# speedup vs baseline: 3.6599x; 3.5243x over previous
import functools

import jax
import jax.numpy as jnp
from jax import lax
from jax.experimental import pallas as pl
from jax.experimental.pallas import tpu as pltpu
from jax.experimental.pallas import tpu_sc as plsc

NN = 10000
EE = 320000
DD = 128
OO = 10
NP = 10240
NC, NS, LN = 2, 16, 16
NW = NC * NS
CH = 128
JC = 80
TOTC = NW * JC
EP = TOTC * CH
CR = CH
JC0 = 80
JC1 = 80
RPT = NP // NS
NBUF = 2

_mesh = plsc.VectorSubcoreMesh(
    core_axis_name="c", subcore_axis_name="s", num_cores=NC, num_subcores=NS)


_RCH = [(k * CR, CR) for k in range(RPT // CR)]
if RPT % CR:
    _RCH.append((RPT - RPT % CR, RPT % CR))


@functools.partial(
    pl.kernel,
    out_type=jax.ShapeDtypeStruct((NC, NP, DD), jnp.float32),
    mesh=_mesh,
    scratch_types=[
        pltpu.VMEM_SHARED((NP, DD), jnp.float32),
        pltpu.VMEM((JC, CH), jnp.int32),
        pltpu.VMEM((CR, DD), jnp.float32),
        pltpu.SemaphoreType.DMA,
    ],
)
def _sc_degree(dst2, deg_out, dacc, dstv, ones, sem):
    c = lax.axis_index("c")
    t = lax.axis_index("s")
    w = c * NS + t

    def _z(r, _):
        for k in range(DD // LN):
            ones[r, pl.ds(k * LN, LN)] = jnp.zeros((LN,), jnp.float32)
        return _
    lax.fori_loop(0, CR, _z, None)
    zh = [pltpu.async_copy(ones.at[pl.ds(0, n)],
                           dacc.at[pl.ds(t * RPT + o, n)], sem)
          for o, n in _RCH]
    for h in zh:
        h.wait()

    def _o(r, _):
        for k in range(DD // LN):
            ones[r, pl.ds(k * LN, LN)] = jnp.ones((LN,), jnp.float32)
        return _
    lax.fori_loop(0, CR, _o, None)
    plsc.subcore_barrier()

    pltpu.sync_copy(dst2.at[pl.ds(w * JC, JC)], dstv)
    handles = [
        pltpu.async_copy(ones, dacc.at[dstv.at[j]], sem, add=True)
        for j in range(JC)
    ]
    for h in handles:
        h.wait()
    plsc.subcore_barrier()

    dh = [pltpu.async_copy(dacc.at[pl.ds(t * RPT + o, n)],
                           deg_out.at[c, pl.ds(t * RPT + o, n)], sem)
          for o, n in _RCH]
    for h in dh:
        h.wait()


IB = 4
JCM = max(JC0, JC1)


@functools.partial(
    pl.kernel,
    out_type=jax.ShapeDtypeStruct((NC, NP, DD), jnp.float32),
    mesh=_mesh,
    scratch_types=[
        pltpu.VMEM_SHARED((NP, DD), jnp.float32),
        pltpu.VMEM((JCM, CH), jnp.int32),
        pltpu.VMEM((IB, CH), jnp.int32),
        [pltpu.VMEM((CH, DD), jnp.float32) for _ in range(2)],
        [pltpu.SemaphoreType.DMA for _ in range(2)],
        [pltpu.SemaphoreType.DMA for _ in range(IB)],
        [pltpu.SemaphoreType.DMA for _ in range(2)],
        pltpu.SemaphoreType.DMA,
    ],
)
def _sc_aggregate(table, srcf, dst2, out, acc, dstv, ibuf, bufs, gsems, isems,
                  ssems, dsem):
    c = lax.axis_index("c")
    t = lax.axis_index("s")

    def _z(r, _):
        for k in range(DD // LN):
            bufs[0][r, pl.ds(k * LN, LN)] = jnp.zeros((LN,), jnp.float32)
        return _
    lax.fori_loop(0, CR, _z, None)
    zh = [pltpu.async_copy(bufs[0].at[pl.ds(0, n)],
                           acc.at[pl.ds(t * RPT + o, n)], dsem)
          for o, n in _RCH]
    for h in zh:
        h.wait()
    plsc.subcore_barrier()

    def _run(jc, base):
        pltpu.sync_copy(dst2.at[pl.ds(base, jc)], dstv.at[pl.ds(0, jc)])

        def _idx_copy(j):
            return pltpu.async_copy(srcf.at[pl.ds((base + j) * CH, CH)],
                                    ibuf.at[j % IB], isems[j % IB])

        def _gather(j):
            return pltpu.async_copy(table.at[ibuf.at[j % IB]],
                                    bufs[j % 2], gsems[j % 2])

        def _scatter(j):
            return pltpu.async_copy(bufs[j % 2], acc.at[dstv.at[j]],
                                    ssems[j % 2], add=True)

        ih, gh, sh = {}, {}, {}
        for j in range(min(IB, jc)):
            ih[j] = _idx_copy(j)
        ih[0].wait()
        gh[0] = _gather(0)
        for j in range(jc):
            if j > 0:
                sh[j - 1].wait()
            nj = j + 1
            if nj < jc:
                ih[nj].wait()
                gh[nj] = _gather(nj)
            gh[j].wait()
            if j + IB < jc:
                ih[j + IB] = _idx_copy(j + IB)
            sh[j] = _scatter(j)
        sh[jc - 1].wait()

    _run(JC0, (c * NS + t) * JC0)

    plsc.subcore_barrier()

    dh = [pltpu.async_copy(acc.at[pl.ds(t * RPT + o, n)],
                           out.at[c, pl.ds(t * RPT + o, n)], dsem)
          for o, n in _RCH]
    for h in dh:
        h.wait()


_GRID = 8
_BR = NP // _GRID


def _row_spec():
    return pl.BlockSpec((_BR, DD), lambda i: (i, 0))


def _full_spec(shape):
    return pl.BlockSpec(shape, lambda i: tuple(0 for _ in shape))


def _tc_pre_body(x_ref, w_ref, deg_ref, o_ref, dinv_ref):
    i = pl.program_id(0)
    deg = 1.0 + deg_ref[0] + deg_ref[1]
    row = i * _BR + jax.lax.broadcasted_iota(jnp.int32, (_BR, DD), 0)
    dinv = jnp.where(row < NN, jax.lax.rsqrt(deg), 0.0)
    dinv_ref[...] = dinv
    o_ref[...] = dinv * jnp.dot(
        x_ref[...], w_ref[...], preferred_element_type=jnp.float32)


def _tc_pre(xp, w0, degp):
    return pl.pallas_call(
        _tc_pre_body,
        grid=(_GRID,),
        in_specs=[
            _row_spec(),
            _full_spec((DD, DD)),
            pl.BlockSpec((NC, _BR, DD), lambda i: (0, i, 0)),
        ],
        out_specs=(_row_spec(), _row_spec()),
        out_shape=(jax.ShapeDtypeStruct((NP, DD), jnp.float32),
                   jax.ShapeDtypeStruct((NP, DD), jnp.float32)),
    )(xp, w0, degp)


def _tc_mid_body(h_ref, s_ref, dinv_ref, b_ref, w_ref, o_ref):
    dinv = dinv_ref[...]
    u = jax.nn.relu(b_ref[...] + dinv * (h_ref[...] + s_ref[0] + s_ref[1]))
    o_ref[...] = dinv * jnp.dot(u, w_ref[...],
                                preferred_element_type=jnp.float32)


def _tc_mid(hp, s, dinv, b, w):
    return pl.pallas_call(
        _tc_mid_body,
        grid=(_GRID,),
        in_specs=[
            _row_spec(),
            pl.BlockSpec((NC, _BR, DD), lambda i: (0, i, 0)),
            _row_spec(),
            _full_spec((1, DD)),
            _full_spec((DD, DD)),
        ],
        out_specs=_row_spec(),
        out_shape=jax.ShapeDtypeStruct((NP, DD), jnp.float32),
    )(hp, s, dinv, b, w)


def _tc_post_body(h_ref, s_ref, dinv_ref, b_ref, w_ref, bl_ref, o_ref):
    u = jax.nn.relu(
        b_ref[...] + dinv_ref[...] * (h_ref[...] + s_ref[0] + s_ref[1]))
    o_ref[...] = jnp.dot(u, w_ref[...],
                         preferred_element_type=jnp.float32) + bl_ref[...]


def _tc_post(hp, s, dinv, b, w, bl):
    return pl.pallas_call(
        _tc_post_body,
        grid=(_GRID,),
        in_specs=[
            _row_spec(),
            pl.BlockSpec((NC, _BR, DD), lambda i: (0, i, 0)),
            _row_spec(),
            _full_spec((1, DD)),
            _full_spec((DD, DD)),
            _full_spec((1, DD)),
        ],
        out_specs=_row_spec(),
        out_shape=jax.ShapeDtypeStruct((NP, DD), jnp.float32),
    )(hp, s, dinv, b, w, bl)


def kernel(x, edge_index, W0, b0, W1, b1, W2, b2, Wlin, blin):
    src = edge_index[0].astype(jnp.int32)
    dst = edge_index[1].astype(jnp.int32)
    pad = NN + (jnp.arange(EP - EE, dtype=jnp.int32) % (NP - NN))
    srcf = jnp.concatenate([src, pad])
    dst2 = jnp.concatenate([dst, pad]).reshape(TOTC, CH)
    xp = jnp.pad(x, ((0, NP - NN), (0, 0)))
    wlp = jnp.pad(Wlin, ((0, 0), (0, DD - OO)))
    blp = jnp.pad(blin, (0, DD - OO)).reshape(1, DD)

    degp = _sc_degree(dst2)
    h0, dinv = _tc_pre(xp, W0, degp)
    s0 = _sc_aggregate(h0, srcf, dst2)
    h1 = _tc_mid(h0, s0, dinv, b0.reshape(1, DD), W1)
    s1 = _sc_aggregate(h1, srcf, dst2)
    h2 = _tc_mid(h1, s1, dinv, b1.reshape(1, DD), W2)
    s2 = _sc_aggregate(h2, srcf, dst2)
    outp = _tc_post(h2, s2, dinv, b2.reshape(1, DD), wlp, blp)
    return outp[:NN, :OO]


# speedup vs baseline: 3.6649x; 1.0013x over previous
import functools

import jax
import jax.numpy as jnp
from jax import lax
from jax.experimental import pallas as pl
from jax.experimental.pallas import tpu as pltpu
from jax.experimental.pallas import tpu_sc as plsc

NN = 10000
EE = 320000
DD = 128
OO = 10
NP = 10240
NC, NS, LN = 2, 16, 16
NW = NC * NS
CH = 128
JC = 80
TOTC = NW * JC
EP = TOTC * CH
CR = CH
JC0 = 80
JC1 = 80
RPT = NP // NS
NBUF = 2

_mesh = plsc.VectorSubcoreMesh(
    core_axis_name="c", subcore_axis_name="s", num_cores=NC, num_subcores=NS)


_RCH = [(k * CR, CR) for k in range(RPT // CR)]
if RPT % CR:
    _RCH.append((RPT - RPT % CR, RPT % CR))


@functools.partial(
    pl.kernel,
    out_type=jax.ShapeDtypeStruct((NC, NP, DD), jnp.float32),
    mesh=_mesh,
    scratch_types=[
        pltpu.VMEM_SHARED((NP, DD), jnp.float32),
        pltpu.VMEM((JC, CH), jnp.int32),
        pltpu.VMEM((CR, DD), jnp.float32),
        pltpu.SemaphoreType.DMA,
    ],
)
def _sc_degree(dst2, deg_out, dacc, dstv, ones, sem):
    c = lax.axis_index("c")
    t = lax.axis_index("s")
    w = c * NS + t

    def _z(r, _):
        for k in range(DD // LN):
            ones[r, pl.ds(k * LN, LN)] = jnp.zeros((LN,), jnp.float32)
        return _
    lax.fori_loop(0, CR, _z, None)
    zh = [pltpu.async_copy(ones.at[pl.ds(0, n)],
                           dacc.at[pl.ds(t * RPT + o, n)], sem)
          for o, n in _RCH]
    for h in zh:
        h.wait()

    def _o(r, _):
        for k in range(DD // LN):
            ones[r, pl.ds(k * LN, LN)] = jnp.ones((LN,), jnp.float32)
        return _
    lax.fori_loop(0, CR, _o, None)
    plsc.subcore_barrier()

    pltpu.sync_copy(dst2.at[pl.ds(w * JC, JC)], dstv)
    handles = [
        pltpu.async_copy(ones, dacc.at[dstv.at[j]], sem, add=True)
        for j in range(JC)
    ]
    for h in handles:
        h.wait()
    plsc.subcore_barrier()

    dh = [pltpu.async_copy(dacc.at[pl.ds(t * RPT + o, n)],
                           deg_out.at[c, pl.ds(t * RPT + o, n)], sem)
          for o, n in _RCH]
    for h in dh:
        h.wait()


IB = 4
JCM = max(JC0, JC1)


@functools.partial(
    pl.kernel,
    out_type=jax.ShapeDtypeStruct((NC, NP, DD), jnp.float32),
    mesh=_mesh,
    scratch_types=[
        pltpu.VMEM_SHARED((NP, DD), jnp.float32),
        pltpu.VMEM((JCM, CH), jnp.int32),
        pltpu.VMEM((IB, CH), jnp.int32),
        [pltpu.VMEM((CH, DD), jnp.float32) for _ in range(2)],
        [pltpu.SemaphoreType.DMA for _ in range(2)],
        [pltpu.SemaphoreType.DMA for _ in range(IB)],
        [pltpu.SemaphoreType.DMA for _ in range(2)],
        pltpu.SemaphoreType.DMA,
    ],
)
def _sc_aggregate(table, srcf, dst2, out, acc, dstv, ibuf, bufs, gsems, isems,
                  ssems, dsem):
    c = lax.axis_index("c")
    t = lax.axis_index("s")

    def _z(r, _):
        for k in range(DD // LN):
            bufs[0][r, pl.ds(k * LN, LN)] = jnp.zeros((LN,), jnp.float32)
        return _
    lax.fori_loop(0, CR, _z, None)
    zh = [pltpu.async_copy(bufs[0].at[pl.ds(0, n)],
                           acc.at[pl.ds(t * RPT + o, n)], dsem)
          for o, n in _RCH]
    for h in zh:
        h.wait()
    plsc.subcore_barrier()

    def _run(jc, base):
        pltpu.sync_copy(dst2.at[pl.ds(base, jc)], dstv.at[pl.ds(0, jc)])

        def _idx_copy(j):
            return pltpu.async_copy(srcf.at[pl.ds((base + j) * CH, CH)],
                                    ibuf.at[j % IB], isems[j % IB])

        def _gather(j):
            return pltpu.async_copy(table.at[ibuf.at[j % IB]],
                                    bufs[j % 2], gsems[j % 2])

        def _scatter(j):
            return pltpu.async_copy(bufs[j % 2], acc.at[dstv.at[j]],
                                    ssems[j % 2], add=True)

        ih, gh, sh = {}, {}, {}
        for j in range(min(IB, jc)):
            ih[j] = _idx_copy(j)
        ih[0].wait()
        gh[0] = _gather(0)
        for j in range(jc):
            if j > 0:
                sh[j - 1].wait()
            nj = j + 1
            if nj < jc:
                ih[nj].wait()
                gh[nj] = _gather(nj)
            gh[j].wait()
            if j + IB < jc:
                ih[j + IB] = _idx_copy(j + IB)
            sh[j] = _scatter(j)
        sh[jc - 1].wait()

    _run(JC0, (c * NS + t) * JC0)

    plsc.subcore_barrier()

    dh = [pltpu.async_copy(acc.at[pl.ds(t * RPT + o, n)],
                           out.at[c, pl.ds(t * RPT + o, n)], dsem)
          for o, n in _RCH]
    for h in dh:
        h.wait()


_GRID = 8
_BR = NP // _GRID


def _row_spec():
    return pl.BlockSpec((_BR, DD), lambda i: (i, 0))


def _full_spec(shape):
    return pl.BlockSpec(shape, lambda i: tuple(0 for _ in shape))


def _tc_matmul_body(x_ref, w_ref, o_ref):
    o_ref[...] = jnp.dot(x_ref[...], w_ref[...],
                         preferred_element_type=jnp.float32)


def _tc_matmul(xp, w0):
    return pl.pallas_call(
        _tc_matmul_body,
        grid=(_GRID,),
        in_specs=[_row_spec(), _full_spec((DD, DD))],
        out_specs=_row_spec(),
        out_shape=jax.ShapeDtypeStruct((NP, DD), jnp.float32),
    )(xp, w0)


def _tc_pre_body(t_ref, deg_ref, o_ref, dinv_ref):
    i = pl.program_id(0)
    deg = 1.0 + deg_ref[0] + deg_ref[1]
    row = i * _BR + jax.lax.broadcasted_iota(jnp.int32, (_BR, DD), 0)
    dinv = jnp.where(row < NN, jax.lax.rsqrt(deg), 0.0)
    dinv_ref[...] = dinv
    o_ref[...] = dinv * t_ref[...]


def _tc_pre(t0, degp):
    return pl.pallas_call(
        _tc_pre_body,
        grid=(_GRID,),
        in_specs=[
            _row_spec(),
            pl.BlockSpec((NC, _BR, DD), lambda i: (0, i, 0)),
        ],
        out_specs=(_row_spec(), _row_spec()),
        out_shape=(jax.ShapeDtypeStruct((NP, DD), jnp.float32),
                   jax.ShapeDtypeStruct((NP, DD), jnp.float32)),
    )(t0, degp)


def _tc_mid_body(h_ref, s_ref, dinv_ref, b_ref, w_ref, o_ref):
    dinv = dinv_ref[...]
    u = jax.nn.relu(b_ref[...] + dinv * (h_ref[...] + s_ref[0] + s_ref[1]))
    o_ref[...] = dinv * jnp.dot(u, w_ref[...],
                                preferred_element_type=jnp.float32)


def _tc_mid(hp, s, dinv, b, w):
    return pl.pallas_call(
        _tc_mid_body,
        grid=(_GRID,),
        in_specs=[
            _row_spec(),
            pl.BlockSpec((NC, _BR, DD), lambda i: (0, i, 0)),
            _row_spec(),
            _full_spec((1, DD)),
            _full_spec((DD, DD)),
        ],
        out_specs=_row_spec(),
        out_shape=jax.ShapeDtypeStruct((NP, DD), jnp.float32),
    )(hp, s, dinv, b, w)


def _tc_post_body(h_ref, s_ref, dinv_ref, b_ref, w_ref, bl_ref, o_ref):
    u = jax.nn.relu(
        b_ref[...] + dinv_ref[...] * (h_ref[...] + s_ref[0] + s_ref[1]))
    o_ref[...] = jnp.dot(u, w_ref[...],
                         preferred_element_type=jnp.float32) + bl_ref[...]


def _tc_post(hp, s, dinv, b, w, bl):
    return pl.pallas_call(
        _tc_post_body,
        grid=(_GRID,),
        in_specs=[
            _row_spec(),
            pl.BlockSpec((NC, _BR, DD), lambda i: (0, i, 0)),
            _row_spec(),
            _full_spec((1, DD)),
            _full_spec((DD, DD)),
            _full_spec((1, DD)),
        ],
        out_specs=_row_spec(),
        out_shape=jax.ShapeDtypeStruct((NP, DD), jnp.float32),
    )(hp, s, dinv, b, w, bl)


def kernel(x, edge_index, W0, b0, W1, b1, W2, b2, Wlin, blin):
    src = edge_index[0].astype(jnp.int32)
    dst = edge_index[1].astype(jnp.int32)
    pad = NN + (jnp.arange(EP - EE, dtype=jnp.int32) % (NP - NN))
    srcf = jnp.concatenate([src, pad])
    dst2 = jnp.concatenate([dst, pad]).reshape(TOTC, CH)
    xp = jnp.pad(x, ((0, NP - NN), (0, 0)))
    wlp = jnp.pad(Wlin, ((0, 0), (0, DD - OO)))
    blp = jnp.pad(blin, (0, DD - OO)).reshape(1, DD)

    t0 = _tc_matmul(xp, W0)
    degp = _sc_degree(dst2)
    h0, dinv = _tc_pre(t0, degp)
    s0 = _sc_aggregate(h0, srcf, dst2)
    h1 = _tc_mid(h0, s0, dinv, b0.reshape(1, DD), W1)
    s1 = _sc_aggregate(h1, srcf, dst2)
    h2 = _tc_mid(h1, s1, dinv, b1.reshape(1, DD), W2)
    s2 = _sc_aggregate(h2, srcf, dst2)
    outp = _tc_post(h2, s2, dinv, b2.reshape(1, DD), wlp, blp)
    return outp[:NN, :OO]
